# baseline (speedup 1.0000x reference)
import functools

import jax
import jax.numpy as jnp
from jax import lax
from jax.experimental import pallas as pl
from jax.experimental.pallas import tpu as pltpu
from jax.experimental.pallas import tpu_sc as plsc

B = 16384
F = 26
D = 16
NCORES = 2
NSUB = 16
NWORK = NCORES * NSUB
ROWS_PER_W = B // NWORK
C = 128
NCHUNK = ROWS_PER_W // C


def _sc_gather_reduce(idx_t, emb_t, fc_flat):
    mesh = plsc.VectorSubcoreMesh(core_axis_name="c", subcore_axis_name="s",
                                  num_cores=NCORES, num_subcores=NSUB)

    @functools.partial(
        pl.kernel,
        out_type=(
            jax.ShapeDtypeStruct((D, B), jnp.float32),
            jax.ShapeDtypeStruct((D, B), jnp.float32),
            jax.ShapeDtypeStruct((B,), jnp.float32),
        ),
        mesh=mesh,
        scratch_types=[
            pltpu.VMEM((F * C,), jnp.int32),
            pltpu.VMEM((D, F * C), jnp.float32),
            pltpu.VMEM((F * C,), jnp.float32),
            pltpu.VMEM((D, C), jnp.float32),
            pltpu.VMEM((D, C), jnp.float32),
            pltpu.VMEM((C,), jnp.float32),
            pltpu.SemaphoreType.DMA,
        ],
        compiler_params=pltpu.CompilerParams(use_tc_tiling_on_sc=False),
    )
    def k(idx_hbm, emb_hbm, fc_hbm, se_hbm, sse_hbm, fcs_hbm,
          idx_v, pd, fc_v, se_b, sse_b, fcs_b, sem):
        w = lax.axis_index("s") * NCORES + lax.axis_index("c")

        def chunk_body(ci, carry):
            base = w * ROWS_PER_W + ci * C
            icps = [pltpu.async_copy(
                idx_hbm.at[f, pl.ds(base, C)],
                idx_v.at[pl.ds(f * C, C)], sem) for f in range(F)]
            for cp in icps:
                cp.wait()
            cps = [pltpu.async_copy(fc_hbm.at[idx_v], fc_v, sem)]
            for d in range(D):
                cps.append(pltpu.async_copy(
                    emb_hbm.at[d].at[idx_v], pd.at[d], sem))
            for cp in cps:
                cp.wait()

            def d_reduce(d, carry2):
                for g in range(C // 16):
                    sl0 = pl.ds(g * 16, 16)
                    acc = pd[d, pl.ds(g * 16, 16)]
                    accq = acc * acc
                    for f in range(1, F):
                        v = pd[d, pl.ds(f * C + g * 16, 16)]
                        acc = acc + v
                        accq = accq + v * v
                    se_b[d, sl0] = acc
                    sse_b[d, sl0] = accq
                return carry2

            lax.fori_loop(0, D, d_reduce, 0, unroll=False)

            for g in range(C // 16):
                sl0 = pl.ds(g * 16, 16)
                acc = fc_v[pl.ds(g * 16, 16)]
                for f in range(1, F):
                    acc = acc + fc_v[pl.ds(f * C + g * 16, 16)]
                fcs_b[sl0] = acc

            pltpu.sync_copy(se_b, se_hbm.at[:, pl.ds(base, C)])
            pltpu.sync_copy(sse_b, sse_hbm.at[:, pl.ds(base, C)])
            pltpu.sync_copy(fcs_b, fcs_hbm.at[pl.ds(base, C)])
            return carry

        lax.fori_loop(0, NCHUNK, chunk_body, 0, unroll=False)

    return k(idx_t, emb_t, fc_flat)


def _tc_dense_t(xn_t, se_t, sse_t, fcs, Wnl, bnl, num_weight, Wfm, bfm,
                cat_bias, *, interpret=False):
    BT = 2048
    grid = (B // BT,)

    def body(xn_ref, se_ref, sse_ref, fcs_ref, wnl_ref, nw_ref, wfm_ref,
             bias_ref, out_ref):
        xn = xn_ref[...]
        hi = jax.lax.Precision.HIGHEST
        nw = nw_ref[...]
        sn = jnp.dot(nw, xn, preferred_element_type=jnp.float32,
                     precision=hi)
        ssn = jnp.dot(nw * nw, xn * xn,
                      preferred_element_type=jnp.float32, precision=hi)
        s = sn + se_ref[...]
        ss = ssn + sse_ref[...]
        fm = 0.5 * (s * s - ss)
        lin = jnp.dot(wnl_ref[...], xn, preferred_element_type=jnp.float32,
                      precision=hi)
        out = (lin + fcs_ref[...]
               + jnp.dot(wfm_ref[...], fm, preferred_element_type=jnp.float32,
                         precision=hi)
               + bias_ref[...])
        out_ref[...] = out

    nw_t = num_weight.reshape(13, D).T
    wnl_t = Wnl.reshape(1, 13)
    wfm_t = Wfm.reshape(1, D)
    bias = (bnl + cat_bias + bfm).reshape(1, 1)
    return pl.pallas_call(
        body,
        grid=grid,
        in_specs=[
            pl.BlockSpec((13, BT), lambda i: (0, i)),
            pl.BlockSpec((D, BT), lambda i: (0, i)),
            pl.BlockSpec((D, BT), lambda i: (0, i)),
            pl.BlockSpec((1, BT), lambda i: (0, i)),
            pl.BlockSpec((1, 13), lambda i: (0, 0)),
            pl.BlockSpec((D, 13), lambda i: (0, 0)),
            pl.BlockSpec((1, D), lambda i: (0, 0)),
            pl.BlockSpec((1, 1), lambda i: (0, 0)),
        ],
        out_specs=pl.BlockSpec((1, BT), lambda i: (0, i)),
        out_shape=jax.ShapeDtypeStruct((1, B), jnp.float32),
        interpret=interpret,
    )(xn_t, se_t, sse_t, fcs.reshape(1, B), wnl_t, nw_t, wfm_t, bias)


def kernel(x_cat, x_num, offsets, emb_table, fc_table, cat_bias, Wnl, bnl,
           num_weight, Wfm, bfm):
    idx_t = (x_cat + offsets[None, :]).T.astype(jnp.int32)
    fc_flat = fc_table.reshape(-1)
    se_t, sse_t, fcs = _sc_gather_reduce(idx_t, emb_table.T, fc_flat)
    out_t = _tc_dense_t(x_num.T, se_t, sse_t, fcs, Wnl, bnl, num_weight,
                        Wfm, bfm, cat_bias)
    return out_t.reshape(B, 1)


# speedup vs baseline: 3.7006x; 3.7006x over previous
import functools

import jax
import jax.numpy as jnp
from jax import lax
from jax.experimental import pallas as pl
from jax.experimental.pallas import tpu as pltpu
from jax.experimental.pallas import tpu_sc as plsc

B = 16384
F = 26
D = 16
NCORES = 2
NSUB = 16
NWORK = NCORES * NSUB
ROWS_PER_W = B // NWORK
C = 128
NCHUNK = ROWS_PER_W // C


def _sc_gather_reduce(idx_t, emb_table, fc_flat, *, interpret=False):
    mesh = plsc.VectorSubcoreMesh(core_axis_name="c", subcore_axis_name="s",
                                  num_cores=NCORES, num_subcores=NSUB)

    @functools.partial(
        pl.kernel,
        out_type=(
            jax.ShapeDtypeStruct((B, D), jnp.float32),
            jax.ShapeDtypeStruct((B, D), jnp.float32),
            jax.ShapeDtypeStruct((B,), jnp.float32),
        ),
        mesh=mesh,
        scratch_types=[
            pltpu.VMEM((F, C), jnp.int32),
            pltpu.VMEM((F, C, D), jnp.float32),
            pltpu.VMEM((F, C), jnp.float32),
            pltpu.VMEM((C, D), jnp.float32),
            pltpu.VMEM((C, D), jnp.float32),
            pltpu.VMEM((C,), jnp.float32),
            pltpu.SemaphoreType.DMA,
        ],
        compiler_params=pltpu.CompilerParams(use_tc_tiling_on_sc=False),
        interpret=interpret,
    )
    def k(idx_hbm, emb_hbm, fc_hbm, se_hbm, sse_hbm, fcs_hbm,
          idx_v, rows_v, fc_v, se_b, sse_b, fcs_b, sem):
        w = lax.axis_index("s") * NCORES + lax.axis_index("c")

        def chunk_body(ci, carry):
            base = w * ROWS_PER_W + ci * C
            pltpu.sync_copy(idx_hbm.at[:, pl.ds(base, C)], idx_v)
            cps = []
            for f in range(F):
                cps.append(pltpu.async_copy(
                    emb_hbm.at[idx_v.at[f]], rows_v.at[f], sem))
                cps.append(pltpu.async_copy(
                    fc_hbm.at[idx_v.at[f]], fc_v.at[f], sem))
            for cp in cps:
                cp.wait()

            def row_body(r, carry2):
                acc_s = rows_v[0, r, :]
                acc_q = acc_s * acc_s
                for f in range(1, F):
                    v = rows_v[f, r, :]
                    acc_s = acc_s + v
                    acc_q = acc_q + v * v
                se_b[r, :] = acc_s
                sse_b[r, :] = acc_q
                return carry2

            lax.fori_loop(0, C, row_body, 0, unroll=False)

            for g in range(C // 16):
                acc = fc_v[0, pl.ds(g * 16, 16)]
                for f in range(1, F):
                    acc = acc + fc_v[f, pl.ds(g * 16, 16)]
                fcs_b[pl.ds(g * 16, 16)] = acc

            pltpu.sync_copy(se_b, se_hbm.at[pl.ds(base, C)])
            pltpu.sync_copy(sse_b, sse_hbm.at[pl.ds(base, C)])
            pltpu.sync_copy(fcs_b, fcs_hbm.at[pl.ds(base, C)])
            return carry

        lax.fori_loop(0, NCHUNK, chunk_body, 0, unroll=False)

    return k(idx_t, emb_table, fc_flat)


def _tc_dense(x_num, se, sse, fcs, Wnl, bnl, num_weight, Wfm, bfm, cat_bias,
              *, interpret=False):
    BT = 2048
    grid = (B // BT,)

    def body(xn_ref, se_ref, sse_ref, fcs_ref, wnl_ref, nw_ref, wfm_ref,
             bias_ref, out_ref):
        xn = xn_ref[...]
        hi = jax.lax.Precision.HIGHEST
        sn = jnp.dot(xn, nw_ref[...], preferred_element_type=jnp.float32,
                     precision=hi)
        ssn = jnp.dot(xn * xn, nw_ref[...] * nw_ref[...],
                      preferred_element_type=jnp.float32, precision=hi)
        s = sn + se_ref[...]
        ss = ssn + sse_ref[...]
        fm = 0.5 * (s * s - ss)
        lin = jnp.dot(xn, wnl_ref[...], preferred_element_type=jnp.float32,
                      precision=hi)
        out = (lin + fcs_ref[...]
               + jnp.dot(fm, wfm_ref[...], preferred_element_type=jnp.float32,
                         precision=hi)
               + bias_ref[...])
        out_ref[...] = out

    nw = num_weight.reshape(13, D)
    bias = (bnl + cat_bias + bfm).reshape(1, 1)
    return pl.pallas_call(
        body,
        grid=grid,
        in_specs=[
            pl.BlockSpec((BT, 13), lambda i: (i, 0)),
            pl.BlockSpec((BT, D), lambda i: (i, 0)),
            pl.BlockSpec((BT, D), lambda i: (i, 0)),
            pl.BlockSpec((BT, 1), lambda i: (i, 0)),
            pl.BlockSpec((13, 1), lambda i: (0, 0)),
            pl.BlockSpec((13, D), lambda i: (0, 0)),
            pl.BlockSpec((D, 1), lambda i: (0, 0)),
            pl.BlockSpec((1, 1), lambda i: (0, 0)),
        ],
        out_specs=pl.BlockSpec((BT, 1), lambda i: (i, 0)),
        out_shape=jax.ShapeDtypeStruct((B, 1), jnp.float32),
        interpret=interpret,
    )(x_num, se, sse, fcs.reshape(B, 1), Wnl, nw, Wfm, bias)


def _tc_relayout(emb_t, *, interpret=False):
    total = emb_t.shape[1]
    cblk = 32768
    grid = (pl.cdiv(total, cblk),)

    def body(in_ref, eye_ref, out_ref):
        y = jax.lax.dot_general(
            in_ref[...], eye_ref[...], (((0,), (0,)), ((), ())),
            preferred_element_type=jnp.float32)
        y8 = y.reshape(cblk // 8, 8, D)
        parts = [
            jax.lax.squeeze(
                jax.lax.slice(y8, (0, e, 0), (cblk // 8, e + 1, D)), (1,))
            for e in range(8)
        ]
        out_ref[...] = jnp.concatenate(parts, axis=1)

    eye = jnp.eye(D, dtype=jnp.float32)
    return pl.pallas_call(
        body,
        grid=grid,
        in_specs=[pl.BlockSpec((D, cblk), lambda i: (0, i)),
                  pl.BlockSpec((D, D), lambda i: (0, 0))],
        out_specs=pl.BlockSpec((cblk // 8, 8 * D), lambda i: (i, 0)),
        out_shape=jax.ShapeDtypeStruct((total // 8, 8 * D), jnp.float32),
        interpret=interpret,
    )(emb_t, eye)


def kernel(x_cat, x_num, offsets, emb_table, fc_table, cat_bias, Wnl, bnl,
           num_weight, Wfm, bfm):
    idx_t = (x_cat + offsets[None, :]).T.astype(jnp.int32)
    fc_flat = fc_table.reshape(-1)
    emb_rm = _tc_relayout(emb_table.T).reshape(emb_table.shape)
    se, sse, fcs = _sc_gather_reduce(idx_t, emb_rm, fc_flat)
    return _tc_dense(x_num, se, sse, fcs, Wnl, bnl, num_weight, Wfm, bfm,
                     cat_bias)


# speedup vs baseline: 3.8889x; 1.0509x over previous
import functools

import jax
import jax.numpy as jnp
from jax import lax
from jax.experimental import pallas as pl
from jax.experimental.pallas import tpu as pltpu
from jax.experimental.pallas import tpu_sc as plsc

B = 16384
F = 26
D = 16
NCORES = 2
NSUB = 16
NWORK = NCORES * NSUB
ROWS_PER_W = B // NWORK
C = 128
NCHUNK = ROWS_PER_W // C


def _sc_gather_reduce(idx_t, emb_table, fc_flat, *, interpret=False):
    mesh = plsc.VectorSubcoreMesh(core_axis_name="c", subcore_axis_name="s",
                                  num_cores=NCORES, num_subcores=NSUB)

    @functools.partial(
        pl.kernel,
        out_type=(
            jax.ShapeDtypeStruct((B, D), jnp.float32),
            jax.ShapeDtypeStruct((B, D), jnp.float32),
            jax.ShapeDtypeStruct((B,), jnp.float32),
        ),
        mesh=mesh,
        scratch_types=[
            pltpu.VMEM((F, C), jnp.int32),
            pltpu.VMEM((F, C, D), jnp.float32),
            pltpu.VMEM((F, C), jnp.float32),
            pltpu.VMEM((C, D), jnp.float32),
            pltpu.VMEM((C, D), jnp.float32),
            pltpu.VMEM((C,), jnp.float32),
            pltpu.SemaphoreType.DMA,
        ],
        compiler_params=pltpu.CompilerParams(use_tc_tiling_on_sc=False),
        interpret=interpret,
    )
    def k(idx_hbm, emb_hbm, fc_hbm, se_hbm, sse_hbm, fcs_hbm,
          idx_v, rows_v, fc_v, se_b, sse_b, fcs_b, sem):
        w = lax.axis_index("s") * NCORES + lax.axis_index("c")

        def chunk_body(ci, carry):
            base = w * ROWS_PER_W + ci * C
            pltpu.sync_copy(idx_hbm.at[:, pl.ds(base, C)], idx_v)
            cps = []
            for f in range(F):
                cps.append(pltpu.async_copy(
                    emb_hbm.at[idx_v.at[f]], rows_v.at[f], sem))
                cps.append(pltpu.async_copy(
                    fc_hbm.at[idx_v.at[f]], fc_v.at[f], sem))
            for cp in cps:
                cp.wait()

            def row_body(r, carry2):
                acc_s = rows_v[0, r, :]
                acc_q = acc_s * acc_s
                for f in range(1, F):
                    v = rows_v[f, r, :]
                    acc_s = acc_s + v
                    acc_q = acc_q + v * v
                se_b[r, :] = acc_s
                sse_b[r, :] = acc_q
                return carry2

            lax.fori_loop(0, C, row_body, 0, unroll=False)

            for g in range(C // 16):
                acc = fc_v[0, pl.ds(g * 16, 16)]
                for f in range(1, F):
                    acc = acc + fc_v[f, pl.ds(g * 16, 16)]
                fcs_b[pl.ds(g * 16, 16)] = acc

            pltpu.sync_copy(se_b, se_hbm.at[pl.ds(base, C)])
            pltpu.sync_copy(sse_b, sse_hbm.at[pl.ds(base, C)])
            pltpu.sync_copy(fcs_b, fcs_hbm.at[pl.ds(base, C)])
            return carry

        lax.fori_loop(0, NCHUNK, chunk_body, 0, unroll=False)

    return k(idx_t, emb_table, fc_flat)


def _tc_dense(x_num, se, sse, fcs, Wnl, bnl, num_weight, Wfm, bfm, cat_bias,
              *, interpret=False):
    BT = 2048
    grid = (B // BT,)

    def body(xn_ref, se_ref, sse_ref, fcs_ref, wnl_ref, nw_ref, wfm_ref,
             bias_ref, out_ref):
        xn = xn_ref[...]
        hi = jax.lax.Precision.HIGHEST
        sn = jnp.dot(xn, nw_ref[...], preferred_element_type=jnp.float32,
                     precision=hi)
        ssn = jnp.dot(xn * xn, nw_ref[...] * nw_ref[...],
                      preferred_element_type=jnp.float32, precision=hi)
        s = sn + se_ref[...]
        ss = ssn + sse_ref[...]
        fm = 0.5 * (s * s - ss)
        lin = jnp.dot(xn, wnl_ref[...], preferred_element_type=jnp.float32,
                      precision=hi)
        out = (lin + fcs_ref[...]
               + jnp.dot(fm, wfm_ref[...], preferred_element_type=jnp.float32,
                         precision=hi)
               + bias_ref[...])
        out_ref[...] = out

    nw = num_weight.reshape(13, D)
    bias = (bnl + cat_bias + bfm).reshape(1, 1)
    return pl.pallas_call(
        body,
        grid=grid,
        in_specs=[
            pl.BlockSpec((BT, 13), lambda i: (i, 0)),
            pl.BlockSpec((BT, D), lambda i: (i, 0)),
            pl.BlockSpec((BT, D), lambda i: (i, 0)),
            pl.BlockSpec((BT, 1), lambda i: (i, 0)),
            pl.BlockSpec((13, 1), lambda i: (0, 0)),
            pl.BlockSpec((13, D), lambda i: (0, 0)),
            pl.BlockSpec((D, 1), lambda i: (0, 0)),
            pl.BlockSpec((1, 1), lambda i: (0, 0)),
        ],
        out_specs=pl.BlockSpec((BT, 1), lambda i: (i, 0)),
        out_shape=jax.ShapeDtypeStruct((B, 1), jnp.float32),
        interpret=interpret,
    )(x_num, se, sse, fcs.reshape(B, 1), Wnl, nw, Wfm, bias)


def _tc_relayout(emb_t, *, interpret=False):
    total = emb_t.shape[1]
    cblk = 32768
    grid = (pl.cdiv(total, cblk),)

    def body(in_ref, eye_ref, out_ref):
        y = jax.lax.dot_general(
            in_ref[...], eye_ref[...], (((0,), (0,)), ((), ())),
            preferred_element_type=jnp.float32)
        y8 = y.reshape(cblk // 8, 8, D)
        parts = [
            jax.lax.squeeze(
                jax.lax.slice(y8, (0, e, 0), (cblk // 8, e + 1, D)), (1,))
            for e in range(8)
        ]
        while len(parts) > 1:
            parts = [jnp.concatenate(parts[i:i + 2], axis=1)
                     for i in range(0, len(parts), 2)]
        out_ref[...] = parts[0]

    eye = jnp.eye(D, dtype=jnp.float32)
    return pl.pallas_call(
        body,
        grid=grid,
        in_specs=[pl.BlockSpec((D, cblk), lambda i: (0, i)),
                  pl.BlockSpec((D, D), lambda i: (0, 0))],
        out_specs=pl.BlockSpec((cblk // 8, 8 * D), lambda i: (i, 0)),
        out_shape=jax.ShapeDtypeStruct((total // 8, 8 * D), jnp.float32),
        interpret=interpret,
    )(emb_t, eye)


def kernel(x_cat, x_num, offsets, emb_table, fc_table, cat_bias, Wnl, bnl,
           num_weight, Wfm, bfm):
    idx_t = (x_cat + offsets[None, :]).T.astype(jnp.int32)
    fc_flat = fc_table.reshape(-1)
    emb_rm = _tc_relayout(emb_table.T).reshape(emb_table.shape)
    se, sse, fcs = _sc_gather_reduce(idx_t, emb_rm, fc_flat)
    return _tc_dense(x_num, se, sse, fcs, Wnl, bnl, num_weight, Wfm, bfm,
                     cat_bias)
